# Initial kernel scaffold; baseline (speedup 1.0000x reference)
import jax
import jax.numpy as jnp
from jax.experimental import pallas as pl


def kernel(sequence_features, timestamps, W0, att_src0, att_dst0, bias0, W1, att_src1, att_dst1, bias1):
    raise NotImplementedError("write your pallas kernel here")


# speedup vs baseline: 127.8666x; 127.8666x over previous
import functools

import jax
import jax.numpy as jnp
from jax.experimental import pallas as pl

_WIN = 5
_SLOPE = 0.2
_HALO = 16
_E1 = 8
_NEG = -1e30


def _fused_gat_body(xpad_ref, w0_ref, a0s_ref, a0d_ref, b0_ref,
                    w1_ref, a1s_ref, a1d_ref, b1_ref, out_ref,
                    *, n, blk, heads, hid):
    i = pl.program_id(0)
    r0 = i * blk
    rows1 = blk + 2 * _E1

    xh = xpad_ref[pl.ds(r0, blk + 2 * _HALO), :]
    xp = jnp.dot(xh, w0_ref[...], preferred_element_type=jnp.float32)

    ncol = heads * hid
    sel = (jax.lax.broadcasted_iota(jnp.int32, (ncol, heads), 0) // hid ==
           jax.lax.broadcasted_iota(jnp.int32, (ncol, heads), 1)
           ).astype(jnp.float32)
    asrc = jnp.dot(xp * a0s_ref[...], sel, preferred_element_type=jnp.float32)
    adst = jnp.dot(xp * a0d_ref[...], sel, preferred_element_type=jnp.float32)

    absrow = (jax.lax.broadcasted_iota(jnp.int32, (rows1, heads), 0)
              + (r0 - _E1))
    adst_c = adst[_E1:_E1 + rows1]
    logits, valids = [], []
    m = jnp.full((rows1, heads), _NEG, dtype=jnp.float32)
    for o in range(-_WIN, _WIN + 1):
        e = asrc[_E1 + o:_E1 + o + rows1] + adst_c
        e = jnp.where(e >= 0, e, _SLOPE * e)
        v = (absrow + o >= 0) & (absrow + o <= n - 1)
        e = jnp.where(v, e, _NEG)
        logits.append(e)
        valids.append(v)
        m = jnp.maximum(m, e)

    num = jnp.zeros((rows1, ncol), dtype=jnp.float32)
    den = jnp.zeros((rows1, heads), dtype=jnp.float32)
    selT = sel.T
    for o, e, v in zip(range(-_WIN, _WIN + 1), logits, valids):
        p = jnp.where(v, jnp.exp(e - m), 0.0)
        den = den + p
        pwide = jnp.dot(p, selT, preferred_element_type=jnp.float32)
        num = num + pwide * xp[_E1 + o:_E1 + o + rows1]
    denw = jnp.dot(den, selT, preferred_element_type=jnp.float32)
    x1 = num / (denw + 1e-16) + b0_ref[...]
    x1 = jnp.where(x1 > 0, x1, jnp.exp(jnp.minimum(x1, 0.0)) - 1.0)

    yp = jnp.dot(x1, w1_ref[...], preferred_element_type=jnp.float32)
    asrc1 = jnp.sum(yp * a1s_ref[...], axis=1, keepdims=True)
    adst1 = jnp.sum(yp * a1d_ref[...], axis=1, keepdims=True)

    absj = jax.lax.broadcasted_iota(jnp.int32, (blk, 1), 0) + r0
    adst1_c = adst1[_E1:_E1 + blk]
    logits2, valids2 = [], []
    m2 = jnp.full((blk, 1), _NEG, dtype=jnp.float32)
    for o in range(-_WIN, _WIN + 1):
        e = asrc1[_E1 + o:_E1 + o + blk] + adst1_c
        e = jnp.where(e >= 0, e, _SLOPE * e)
        v = (absj + o >= 0) & (absj + o <= n - 1)
        e = jnp.where(v, e, _NEG)
        logits2.append(e)
        valids2.append(v)
        m2 = jnp.maximum(m2, e)

    num2 = jnp.zeros((blk, yp.shape[1]), dtype=jnp.float32)
    den2 = jnp.zeros((blk, 1), dtype=jnp.float32)
    for o, e, v in zip(range(-_WIN, _WIN + 1), logits2, valids2):
        p = jnp.where(v, jnp.exp(e - m2), 0.0)
        den2 = den2 + p
        num2 = num2 + p * yp[_E1 + o:_E1 + o + blk]
    out_ref[...] = num2 / (den2 + 1e-16) + b1_ref[...]


def _pick_block(n):
    for b in (1000, 512, 800, 400, 256, 200, 128, 80, 64, 40, 16, 8):
        if n % b == 0:
            return b
    return n


@jax.jit
def kernel(sequence_features, timestamps, W0, att_src0, att_dst0, bias0,
           W1, att_src1, att_dst1, bias1):
    del timestamps
    n, d = sequence_features.shape
    heads, hid = att_src0.shape
    blk = _pick_block(n)

    xpad = jnp.pad(sequence_features, ((_HALO, _HALO), (0, 0)))
    a0s = att_src0.reshape(1, heads * hid)
    a0d = att_dst0.reshape(1, heads * hid)
    a1s = att_src1.reshape(1, -1)
    a1d = att_dst1.reshape(1, -1)
    b0 = bias0.reshape(1, -1)
    b1 = bias1.reshape(1, -1)

    body = functools.partial(_fused_gat_body, n=n, blk=blk,
                             heads=heads, hid=hid)
    full = lambda a: pl.BlockSpec(a.shape, lambda i: (0,) * a.ndim)
    out = pl.pallas_call(
        body,
        grid=(n // blk,),
        in_specs=[full(xpad), full(W0), full(a0s), full(a0d), full(b0),
                  full(W1), full(a1s), full(a1d), full(b1)],
        out_specs=pl.BlockSpec((blk, W1.shape[1]), lambda i: (i, 0)),
        out_shape=jax.ShapeDtypeStruct((n, W1.shape[1]), jnp.float32),
    )(xpad, W0, a0s, a0d, b0, W1, a1s, a1d, b1)
    return out


# speedup vs baseline: 154.5245x; 1.2085x over previous
import functools

import jax
import jax.numpy as jnp
from jax.experimental import pallas as pl

_WIN = 5
_SLOPE = 0.2
_HALO = 16
_E1 = 8
_NEG = -1e30


def _fused_gat_body(xpad_ref, w0_ref, a0s_ref, a0d_ref, b0_ref,
                    w1_ref, a1s_ref, a1d_ref, b1_ref, sel_ref, out_ref,
                    *, n, blk, heads, hid):
    i = pl.program_id(0)
    r0 = i * blk
    rows1 = blk + 2 * _E1
    ncol = heads * hid

    xh = xpad_ref[pl.ds(r0, blk + 2 * _HALO), :]
    xp = jnp.dot(xh, w0_ref[...], preferred_element_type=jnp.float32)

    sel = sel_ref[...]
    selT = sel_ref[...].T
    asrc = jnp.dot(xp * a0s_ref[...], sel, preferred_element_type=jnp.float32)
    adst = jnp.dot(xp * a0d_ref[...], sel, preferred_element_type=jnp.float32)

    def layers(masked):
        adst_c = adst[_E1:_E1 + rows1]
        if masked:
            absrow = (jax.lax.broadcasted_iota(jnp.int32, (rows1, heads), 0)
                      + (r0 - _E1))
        logits = []
        m = jnp.full((rows1, heads), _NEG, dtype=jnp.float32)
        for o in range(-_WIN, _WIN + 1):
            e = asrc[_E1 + o:_E1 + o + rows1] + adst_c
            e = jnp.maximum(e, _SLOPE * e)
            if masked:
                v = (absrow + o >= 0) & (absrow + o <= n - 1)
                e = jnp.where(v, e, _NEG)
            logits.append(e)
            m = jnp.maximum(m, e)

        num = jnp.zeros((rows1, ncol), dtype=jnp.float32)
        den = jnp.zeros((rows1, heads), dtype=jnp.float32)
        for o, e in zip(range(-_WIN, _WIN + 1), logits):
            p = jnp.exp(e - m)
            den = den + p
            pwide = jnp.dot(p, selT, preferred_element_type=jnp.float32)
            num = num + pwide * xp[_E1 + o:_E1 + o + rows1]
        rcp = 1.0 / (den + 1e-16)
        rcpw = jnp.dot(rcp, selT, preferred_element_type=jnp.float32)
        x1 = num * rcpw + b0_ref[...]
        x1 = jnp.maximum(x1, jnp.exp(jnp.minimum(x1, 0.0)) - 1.0)

        yp = jnp.dot(x1, w1_ref[...], preferred_element_type=jnp.float32)
        asrc1 = jnp.sum(yp * a1s_ref[...], axis=1, keepdims=True)
        adst1 = jnp.sum(yp * a1d_ref[...], axis=1, keepdims=True)

        adst1_c = adst1[_E1:_E1 + blk]
        if masked:
            absj = jax.lax.broadcasted_iota(jnp.int32, (blk, 1), 0) + r0
        logits2 = []
        m2 = jnp.full((blk, 1), _NEG, dtype=jnp.float32)
        for o in range(-_WIN, _WIN + 1):
            e = asrc1[_E1 + o:_E1 + o + blk] + adst1_c
            e = jnp.maximum(e, _SLOPE * e)
            if masked:
                v = (absj + o >= 0) & (absj + o <= n - 1)
                e = jnp.where(v, e, _NEG)
            logits2.append(e)
            m2 = jnp.maximum(m2, e)

        num2 = jnp.zeros((blk, yp.shape[1]), dtype=jnp.float32)
        den2 = jnp.zeros((blk, 1), dtype=jnp.float32)
        for o, e in zip(range(-_WIN, _WIN + 1), logits2):
            p = jnp.exp(e - m2)
            den2 = den2 + p
            num2 = num2 + p * yp[_E1 + o:_E1 + o + blk]
        out_ref[...] = num2 * (1.0 / (den2 + 1e-16)) + b1_ref[...]

    interior = jnp.logical_and(r0 - _E1 - _WIN >= 0,
                               r0 + blk + _E1 + _WIN <= n)

    @pl.when(interior)
    def _():
        layers(masked=False)

    @pl.when(jnp.logical_not(interior))
    def _():
        layers(masked=True)


def _pick_block(n):
    for b in (1000, 512, 800, 400, 256, 200, 128, 80, 64, 40, 16, 8):
        if n % b == 0:
            return b
    return n


@jax.jit
def kernel(sequence_features, timestamps, W0, att_src0, att_dst0, bias0,
           W1, att_src1, att_dst1, bias1):
    del timestamps
    n, d = sequence_features.shape
    heads, hid = att_src0.shape
    blk = _pick_block(n)

    xpad = jnp.pad(sequence_features, ((_HALO, _HALO), (0, 0)))
    a0s = att_src0.reshape(1, heads * hid)
    a0d = att_dst0.reshape(1, heads * hid)
    a1s = att_src1.reshape(1, -1)
    a1d = att_dst1.reshape(1, -1)
    b0 = bias0.reshape(1, -1)
    b1 = bias1.reshape(1, -1)
    ncol = heads * hid
    sel = (jnp.arange(ncol)[:, None] // hid ==
           jnp.arange(heads)[None, :]).astype(jnp.float32)

    body = functools.partial(_fused_gat_body, n=n, blk=blk,
                             heads=heads, hid=hid)
    full = lambda a: pl.BlockSpec(a.shape, lambda i: (0,) * a.ndim)
    out = pl.pallas_call(
        body,
        grid=(n // blk,),
        in_specs=[full(xpad), full(W0), full(a0s), full(a0d), full(b0),
                  full(W1), full(a1s), full(a1d), full(b1), full(sel)],
        out_specs=pl.BlockSpec((blk, W1.shape[1]), lambda i: (i, 0)),
        out_shape=jax.ShapeDtypeStruct((n, W1.shape[1]), jnp.float32),
    )(xpad, W0, a0s, a0d, b0, W1, a1s, a1d, b1, sel)
    return out
